# baseline (speedup 1.0000x reference)
import functools

import jax
import jax.numpy as jnp
import numpy as np
from jax import lax
from jax.experimental import pallas as pl
from jax.experimental.pallas import tpu as pltpu
from jax.experimental.pallas import tpu_sc as plsc

N_NODES = 10000
N_EDGES = 320000
D = 128
DH = D // 2
DROPOUT_P = 0.1

NC = 2
NS = 16
L = 16

CHUNK = 256
NBUF = 3
NPHASE = 2
EDGES_PER_TILE = -(-(N_EDGES // NS) // (CHUNK * NBUF * NPHASE)) * (
    CHUNK * NBUF * NPHASE)
NCHUNK = EDGES_PER_TILE // CHUNK
PCHUNK = NCHUNK // NPHASE
E_PAD = EDGES_PER_TILE * NS
ROWS_PER_TILE = N_NODES // NS
ZROWS = 125

_mesh = plsc.VectorSubcoreMesh(core_axis_name="c", subcore_axis_name="s")


def _tf2x32(k1, k2, x0, x1):
    rot = [[13, 15, 26, 6], [17, 29, 16, 24]]
    ks = [np.uint32(k1), np.uint32(k2),
          np.uint32(k1) ^ np.uint32(k2) ^ np.uint32(0x1BD11BDA)]
    x = [(x0 + ks[0]).astype(np.uint32), (x1 + ks[1]).astype(np.uint32)]
    for i in range(5):
        for r in rot[i % 2]:
            x[0] = (x[0] + x[1]).astype(np.uint32)
            x[1] = ((x[1] << np.uint32(r)) |
                    (x[1] >> np.uint32(32 - r))).astype(np.uint32)
            x[1] = x[0] ^ x[1]
        x[0] = (x[0] + ks[(i + 1) % 3]).astype(np.uint32)
        x[1] = (x[1] + ks[(i + 2) % 3] + np.uint32(i + 1)).astype(np.uint32)
    return x


def _keep_mask() -> np.ndarray:
    with np.errstate(over="ignore"):
        fk = _tf2x32(np.uint32(0), np.uint32(0),
                     np.array([0], np.uint32), np.array([12345], np.uint32))
        idx = np.arange(N_NODES * D, dtype=np.uint64)
        hi = (idx >> np.uint64(32)).astype(np.uint32)
        lo = idx.astype(np.uint32)
        b = _tf2x32(fk[0][0], fk[1][0], hi, lo)
        bits = b[0] ^ b[1]
        fb = (bits >> np.uint32(9)) | np.uint32(0x3F800000)
        floats = fb.view(np.float32) - np.float32(1.0)
        keep = floats < np.float32(1.0 - DROPOUT_P)
    return keep.reshape(N_NODES, D).astype(np.float32)


_KEEPF = _keep_mask()


@functools.partial(
    pl.kernel,
    mesh=_mesh,
    out_type=jax.ShapeDtypeStruct((NC, N_NODES, DH), jnp.float32),
    scratch_types=[
        pltpu.VMEM((PCHUNK, CHUNK), jnp.int32),
        pltpu.VMEM((PCHUNK, CHUNK), jnp.int32),
        pltpu.VMEM((PCHUNK, CHUNK), jnp.float32),
        pltpu.VMEM((NBUF, CHUNK, DH), jnp.float32),
        pltpu.VMEM_SHARED((N_NODES, DH), jnp.float32),
        pltpu.SemaphoreType.DMA,
        pltpu.SemaphoreType.DMA,
        pltpu.SemaphoreType.DMA,
        pltpu.SemaphoreType.DMA,
        pltpu.SemaphoreType.DMA,
        pltpu.SemaphoreType.DMA,
    ],
    compiler_params=pltpu.CompilerParams(use_tc_tiling_on_sc=False, needs_layout_passes=False),
)
def _spmm_sc(support_hbm, src_hbm, dst_hbm, w_hbm, out_hbm,
             src_v, dst_v, w_v, rows_v, acc, g0, g1, g2, t0, t1, t2):
    c = lax.axis_index("c")
    s = lax.axis_index("s")
    gsem = (g0, g1, g2)
    ssem = (t0, t1, t2)

    zero = jnp.zeros((L,), jnp.float32)

    def _zrow(i, carry):
        for dd in range(DH // L):
            rows_v[0, i, pl.ds(dd * L, L)] = zero
        return carry

    lax.fori_loop(0, ZROWS, _zrow, 0)
    for r in range(ROWS_PER_TILE // ZROWS):
        pltpu.sync_copy(rows_v.at[0, pl.ds(0, ZROWS)],
                        acc.at[pl.ds(s * ROWS_PER_TILE + r * ZROWS, ZROWS)])

    plsc.subcore_barrier()

    def _gather_start(j, b):
        pltpu.async_copy(support_hbm.at[c].at[src_v.at[j]], rows_v.at[b],
                         gsem[b])

    def _gather_wait(j, b):
        pltpu.make_async_copy(support_hbm.at[c].at[src_v.at[j]], rows_v.at[b],
                              gsem[b]).wait()

    def _scale(j, b):
        jv = jnp.full((L,), j, jnp.int32)

        @plsc.parallel_loop(0, CHUNK, step=4, unroll=2)
        def _sc(k):
            kv = jnp.full((L,), k, jnp.int32)
            for u in range(4):
                e = k + u
                wv = plsc.load_gather(w_v, [jv, kv + u])
                for dd in range(DH // L):
                    sl = pl.ds(dd * L, L)
                    rows_v[b, e, sl] = rows_v[b, e, sl] * wv

    def _scatter_start(j, b):
        pltpu.async_copy(rows_v.at[b], acc.at[dst_v.at[j]], ssem[b], add=True)

    def _scatter_wait(j, b):
        pltpu.make_async_copy(rows_v.at[b], acc.at[dst_v.at[j]],
                              ssem[b]).wait()

    for p in range(NPHASE):
        pltpu.sync_copy(src_hbm.at[s, pl.ds(p * PCHUNK, PCHUNK)], src_v)
        pltpu.sync_copy(dst_hbm.at[s, pl.ds(p * PCHUNK, PCHUNK)], dst_v)
        pltpu.sync_copy(w_hbm.at[s, pl.ds(p * PCHUNK, PCHUNK)], w_v)

        _gather_start(0, 0)
        _gather_start(1, 1)

        def _ring(t, carry):
            j0 = t * NBUF
            for i in range(NBUF):
                j = j0 + i
                b = i
                bz = (i + 2) % NBUF
                if i == 0:
                    @pl.when(j0 > 0)
                    def _():
                        _scatter_wait(j0 - 1, bz)
                else:
                    _scatter_wait(j - 1, bz)

                @pl.when(j + 2 < PCHUNK)
                def _():
                    _gather_start(j + 2, bz)

                _gather_wait(j, b)
                _scale(j, b)
                _scatter_start(j, b)
            return carry

        lax.fori_loop(0, PCHUNK // NBUF, _ring, 0)
        _scatter_wait(PCHUNK - 1, (PCHUNK - 1) % NBUF)

    plsc.subcore_barrier()

    WB = 624
    pltpu.sync_copy(acc.at[pl.ds(s * WB, WB)],
                    out_hbm.at[c, pl.ds(s * WB, WB)])

    @pl.when(s == NS - 1)
    def _tail():
        pltpu.sync_copy(acc.at[pl.ds(NS * WB, N_NODES - NS * WB)],
                        out_hbm.at[c, pl.ds(NS * WB, N_NODES - NS * WB)])


ROW_BLK = 1000


def _mm1_body(h_ref, w_ref, o_ref):
    y = jnp.dot(h_ref[...], w_ref[...], preferred_element_type=jnp.float32)
    o_ref[0] = y[:, :DH]
    o_ref[1] = y[:, DH:]


def _mid_body(p_ref, b_ref, k_ref, w_ref, o_ref):
    x = jnp.concatenate([p_ref[0], p_ref[1]], axis=-1) + b_ref[...]
    x = jnp.where(x >= 0, x, 0.25 * x)
    x = jnp.where(k_ref[...] > 0, x / (1.0 - DROPOUT_P), 0.0)
    y = jnp.dot(x, w_ref[...], preferred_element_type=jnp.float32)
    o_ref[0] = y[:, :DH]
    o_ref[1] = y[:, DH:]


def _fin_body(p_ref, b_ref, o_ref):
    x = jnp.concatenate([p_ref[0], p_ref[1]], axis=-1) + b_ref[...]
    o_ref[...] = jnp.where(x >= 0, x, 0.25 * x)


def kernel(H, edge_index, edge_weight, W1, b1, W2, b2):
    grid = (N_NODES // ROW_BLK,)
    f32 = jnp.float32

    src = edge_index[1].astype(jnp.int32)
    dst = edge_index[0].astype(jnp.int32)
    pad = E_PAD - N_EDGES
    src_p = jnp.concatenate([src, jnp.zeros((pad,), jnp.int32)]
                            ).reshape(NS, NCHUNK, CHUNK)
    dst_p = jnp.concatenate([dst, jnp.zeros((pad,), jnp.int32)]
                            ).reshape(NS, NCHUNK, CHUNK)
    w_p = jnp.concatenate([edge_weight, jnp.zeros((pad,), f32)]
                          ).reshape(NS, NCHUNK, CHUNK)

    keepf = jnp.asarray(_KEEPF)

    b1r = b1.reshape(1, D)
    b2r = b2.reshape(1, D)

    halves = jax.ShapeDtypeStruct((NC, N_NODES, DH), f32)
    half_spec = pl.BlockSpec((NC, ROW_BLK, DH), lambda i: (0, i, 0))

    support = pl.pallas_call(
        _mm1_body,
        grid=grid,
        in_specs=[pl.BlockSpec((ROW_BLK, D), lambda i: (i, 0)),
                  pl.BlockSpec((D, D), lambda i: (0, 0))],
        out_specs=half_spec,
        out_shape=halves,
    )(H, W1)

    part1 = _spmm_sc(support, src_p, dst_p, w_p)

    support2 = pl.pallas_call(
        _mid_body,
        grid=grid,
        in_specs=[half_spec,
                  pl.BlockSpec((1, D), lambda i: (0, 0)),
                  pl.BlockSpec((ROW_BLK, D), lambda i: (i, 0)),
                  pl.BlockSpec((D, D), lambda i: (0, 0))],
        out_specs=half_spec,
        out_shape=halves,
    )(part1, b1r, keepf, W2)

    part2 = _spmm_sc(support2, src_p, dst_p, w_p)

    out2 = pl.pallas_call(
        _fin_body,
        grid=grid,
        in_specs=[half_spec,
                  pl.BlockSpec((1, D), lambda i: (0, 0))],
        out_specs=pl.BlockSpec((ROW_BLK, D), lambda i: (i, 0)),
        out_shape=jax.ShapeDtypeStruct((N_NODES, D), f32),
    )(part2, b2r)

    return out2


# speedup vs baseline: 2.5779x; 2.5779x over previous
import functools

import jax
import jax.numpy as jnp
import numpy as np
from jax import lax
from jax.experimental import pallas as pl
from jax.experimental.pallas import tpu as pltpu
from jax.experimental.pallas import tpu_sc as plsc

N_NODES = 10000
N_EDGES = 320000
D = 128
DH = D // 2
DROPOUT_P = 0.1

NC = 2
NS = 16
L = 16

CHUNK = 128
NBUF = 3
EDGES_PER_TILE = -(-(N_EDGES // NS) // (CHUNK * NBUF)) * CHUNK * NBUF
NCHUNK = EDGES_PER_TILE // CHUNK
E_PAD = EDGES_PER_TILE * NS
ROWS_PER_TILE = N_NODES // NS
ZROWS = 125

_mesh = plsc.VectorSubcoreMesh(core_axis_name="c", subcore_axis_name="s")


def _tf2x32(k1, k2, x0, x1):
    rot = [[13, 15, 26, 6], [17, 29, 16, 24]]
    ks = [np.uint32(k1), np.uint32(k2),
          np.uint32(k1) ^ np.uint32(k2) ^ np.uint32(0x1BD11BDA)]
    x = [(x0 + ks[0]).astype(np.uint32), (x1 + ks[1]).astype(np.uint32)]
    for i in range(5):
        for r in rot[i % 2]:
            x[0] = (x[0] + x[1]).astype(np.uint32)
            x[1] = ((x[1] << np.uint32(r)) |
                    (x[1] >> np.uint32(32 - r))).astype(np.uint32)
            x[1] = x[0] ^ x[1]
        x[0] = (x[0] + ks[(i + 1) % 3]).astype(np.uint32)
        x[1] = (x[1] + ks[(i + 2) % 3] + np.uint32(i + 1)).astype(np.uint32)
    return x


def _keep_mask() -> np.ndarray:
    with np.errstate(over="ignore"):
        fk = _tf2x32(np.uint32(0), np.uint32(0),
                     np.array([0], np.uint32), np.array([12345], np.uint32))
        idx = np.arange(N_NODES * D, dtype=np.uint64)
        hi = (idx >> np.uint64(32)).astype(np.uint32)
        lo = idx.astype(np.uint32)
        b = _tf2x32(fk[0][0], fk[1][0], hi, lo)
        bits = b[0] ^ b[1]
        fb = (bits >> np.uint32(9)) | np.uint32(0x3F800000)
        floats = fb.view(np.float32) - np.float32(1.0)
        keep = floats < np.float32(1.0 - DROPOUT_P)
    return keep.reshape(N_NODES, D).astype(np.float32)


_KEEPF = _keep_mask()


@functools.partial(
    pl.kernel,
    mesh=_mesh,
    out_type=jax.ShapeDtypeStruct((NC, N_NODES, DH), jnp.float32),
    scratch_types=[
        pltpu.VMEM((NCHUNK, CHUNK), jnp.int32),
        pltpu.VMEM((NCHUNK, CHUNK), jnp.int32),
        pltpu.VMEM((NCHUNK, CHUNK), jnp.float32),
        pltpu.VMEM((NBUF, CHUNK, DH), jnp.float32),
        pltpu.VMEM_SHARED((N_NODES, DH), jnp.float32),
        pltpu.SemaphoreType.DMA,
        pltpu.SemaphoreType.DMA,
        pltpu.SemaphoreType.DMA,
        pltpu.SemaphoreType.DMA,
        pltpu.SemaphoreType.DMA,
        pltpu.SemaphoreType.DMA,
    ],
    compiler_params=pltpu.CompilerParams(use_tc_tiling_on_sc=False, needs_layout_passes=False),
)
def _spmm_sc(support_hbm, src_hbm, dst_hbm, w_hbm, out_hbm,
             src_v, dst_v, w_v, rows_v, acc, g0, g1, g2, t0, t1, t2):
    c = lax.axis_index("c")
    s = lax.axis_index("s")
    gsem = (g0, g1, g2)
    ssem = (t0, t1, t2)

    zero = jnp.zeros((L,), jnp.float32)

    def _zrow(i, carry):
        for dd in range(DH // L):
            rows_v[0, i, pl.ds(dd * L, L)] = zero
        return carry

    lax.fori_loop(0, ZROWS, _zrow, 0)
    for r in range(ROWS_PER_TILE // ZROWS):
        pltpu.sync_copy(rows_v.at[0, pl.ds(0, ZROWS)],
                        acc.at[pl.ds(s * ROWS_PER_TILE + r * ZROWS, ZROWS)])

    pltpu.sync_copy(src_hbm.at[s], src_v)
    pltpu.sync_copy(dst_hbm.at[s], dst_v)
    pltpu.sync_copy(w_hbm.at[s], w_v)

    plsc.subcore_barrier()

    def _gather_start(j, b):
        pltpu.async_copy(support_hbm.at[c].at[src_v.at[j]], rows_v.at[b],
                         gsem[b])

    def _gather_wait(j, b):
        pltpu.make_async_copy(support_hbm.at[c].at[src_v.at[j]], rows_v.at[b],
                              gsem[b]).wait()

    def _scale(j, b):
        jv = jnp.full((L,), j, jnp.int32)

        @plsc.parallel_loop(0, CHUNK, step=4, unroll=2)
        def _sc(k):
            kv = jnp.full((L,), k, jnp.int32)
            for u in range(4):
                e = k + u
                wv = plsc.load_gather(w_v, [jv, kv + u])
                for dd in range(DH // L):
                    sl = pl.ds(dd * L, L)
                    rows_v[b, e, sl] = rows_v[b, e, sl] * wv

    def _scatter_start(j, b):
        pltpu.async_copy(rows_v.at[b], acc.at[dst_v.at[j]], ssem[b], add=True)

    def _scatter_wait(j, b):
        pltpu.make_async_copy(rows_v.at[b], acc.at[dst_v.at[j]],
                              ssem[b]).wait()

    _gather_start(0, 0)
    _gather_start(1, 1)

    def _ring(t, carry):
        j0 = t * NBUF
        for i in range(NBUF):
            j = j0 + i
            b = i
            bz = (i + 2) % NBUF
            _gather_wait(j, b)
            _scale(j, b)
            if i == 0:
                @pl.when(j0 > 0)
                def _():
                    _scatter_wait(j0 - 1, bz)
            else:
                _scatter_wait(j - 1, bz)

            @pl.when(j + 2 < NCHUNK)
            def _():
                _gather_start(j + 2, bz)

            _scatter_start(j, b)
        return carry

    lax.fori_loop(0, NCHUNK // NBUF, _ring, 0)
    _scatter_wait(NCHUNK - 1, (NCHUNK - 1) % NBUF)

    plsc.subcore_barrier()

    WB = 624
    pltpu.sync_copy(acc.at[pl.ds(s * WB, WB)],
                    out_hbm.at[c, pl.ds(s * WB, WB)])

    @pl.when(s == NS - 1)
    def _tail():
        pltpu.sync_copy(acc.at[pl.ds(NS * WB, N_NODES - NS * WB)],
                        out_hbm.at[c, pl.ds(NS * WB, N_NODES - NS * WB)])


ROW_BLK = 1000


def _mm1_body(h_ref, w_ref, o_ref):
    y = jnp.dot(h_ref[...], w_ref[...], preferred_element_type=jnp.float32)
    o_ref[0] = y[:, :DH]
    o_ref[1] = y[:, DH:]


def _mid_body(p_ref, b_ref, k_ref, w_ref, o_ref):
    x = jnp.concatenate([p_ref[0], p_ref[1]], axis=-1) + b_ref[...]
    x = jnp.where(x >= 0, x, 0.25 * x)
    x = jnp.where(k_ref[...] > 0, x / (1.0 - DROPOUT_P), 0.0)
    y = jnp.dot(x, w_ref[...], preferred_element_type=jnp.float32)
    o_ref[0] = y[:, :DH]
    o_ref[1] = y[:, DH:]


def _fin_body(p_ref, b_ref, o_ref):
    x = jnp.concatenate([p_ref[0], p_ref[1]], axis=-1) + b_ref[...]
    o_ref[...] = jnp.where(x >= 0, x, 0.25 * x)


def kernel(H, edge_index, edge_weight, W1, b1, W2, b2):
    grid = (N_NODES // ROW_BLK,)
    f32 = jnp.float32

    src = edge_index[1].astype(jnp.int32)
    dst = edge_index[0].astype(jnp.int32)
    pad = E_PAD - N_EDGES
    src_p = jnp.concatenate([src, jnp.zeros((pad,), jnp.int32)]
                            ).reshape(NS, NCHUNK, CHUNK)
    dst_p = jnp.concatenate([dst, jnp.zeros((pad,), jnp.int32)]
                            ).reshape(NS, NCHUNK, CHUNK)
    w_p = jnp.concatenate([edge_weight, jnp.zeros((pad,), f32)]
                          ).reshape(NS, NCHUNK, CHUNK)

    keepf = jnp.asarray(_KEEPF)

    b1r = b1.reshape(1, D)
    b2r = b2.reshape(1, D)

    halves = jax.ShapeDtypeStruct((NC, N_NODES, DH), f32)
    half_spec = pl.BlockSpec((NC, ROW_BLK, DH), lambda i: (0, i, 0))

    support = pl.pallas_call(
        _mm1_body,
        grid=grid,
        in_specs=[pl.BlockSpec((ROW_BLK, D), lambda i: (i, 0)),
                  pl.BlockSpec((D, D), lambda i: (0, 0))],
        out_specs=half_spec,
        out_shape=halves,
    )(H, W1)

    part1 = _spmm_sc(support, src_p, dst_p, w_p)

    support2 = pl.pallas_call(
        _mid_body,
        grid=grid,
        in_specs=[half_spec,
                  pl.BlockSpec((1, D), lambda i: (0, 0)),
                  pl.BlockSpec((ROW_BLK, D), lambda i: (i, 0)),
                  pl.BlockSpec((D, D), lambda i: (0, 0))],
        out_specs=half_spec,
        out_shape=halves,
    )(part1, b1r, keepf, W2)

    part2 = _spmm_sc(support2, src_p, dst_p, w_p)

    out2 = pl.pallas_call(
        _fin_body,
        grid=grid,
        in_specs=[half_spec,
                  pl.BlockSpec((1, D), lambda i: (0, 0))],
        out_specs=pl.BlockSpec((ROW_BLK, D), lambda i: (i, 0)),
        out_shape=jax.ShapeDtypeStruct((N_NODES, D), f32),
    )(part2, b2r)

    return out2
